# Initial kernel scaffold; baseline (speedup 1.0000x reference)
import jax
import jax.numpy as jnp
from jax.experimental import pallas as pl


def kernel(positions, position_encoding):
    raise NotImplementedError("write your pallas kernel here")


# speedup vs baseline: 6.2988x; 6.2988x over previous
import functools

import jax
import jax.numpy as jnp
from jax import lax
from jax.experimental import pallas as pl
from jax.experimental.pallas import tpu as pltpu
from jax.experimental.pallas import tpu_sc as plsc

BATCH = 16384
SEQ = 200
DIM = 16
TOTAL = BATCH * SEQ

_NUM_CORES = 2
_NUM_SUBCORES = 16
_NW = _NUM_CORES * _NUM_SUBCORES
_PER_W = TOTAL // _NW
_CHUNK = 4096
_NCHUNKS = _PER_W // _CHUNK


def _sc_gather(idx_flat, table):
    mesh = plsc.VectorSubcoreMesh(
        core_axis_name="c", subcore_axis_name="s", num_cores=_NUM_CORES
    )

    @functools.partial(
        pl.kernel,
        out_type=jax.ShapeDtypeStruct((TOTAL, DIM), jnp.float32),
        mesh=mesh,
        scratch_types=[
            pltpu.VMEM((_CHUNK,), jnp.int32),
            pltpu.VMEM((_CHUNK, DIM), jnp.float32),
            pltpu.SemaphoreType.DMA,
        ],
        compiler_params=pltpu.CompilerParams(use_tc_tiling_on_sc=False),
    )
    def k(idx_hbm, table_hbm, out_hbm, idx_v, rows_v, sem):
        wid = lax.axis_index("s") * _NUM_CORES + lax.axis_index("c")
        base = wid * _PER_W

        def chunk_body(i, carry):
            off = base + i * _CHUNK
            pltpu.sync_copy(idx_hbm.at[pl.ds(off, _CHUNK)], idx_v)
            pltpu.async_copy(table_hbm.at[idx_v], rows_v, sem).wait()
            pltpu.sync_copy(rows_v, out_hbm.at[pl.ds(off, _CHUNK)])
            return carry

        lax.fori_loop(0, _NCHUNKS, chunk_body, 0)

    return k(idx_flat, table)


def kernel(positions, position_encoding):
    idx_flat = positions.reshape(TOTAL)
    out = _sc_gather(idx_flat, position_encoding)
    return out.reshape(BATCH, SEQ, DIM)


# speedup vs baseline: 6.3300x; 1.0050x over previous
import functools

import jax
import jax.numpy as jnp
from jax import lax
from jax.experimental import pallas as pl
from jax.experimental.pallas import tpu as pltpu
from jax.experimental.pallas import tpu_sc as plsc

BATCH = 16384
SEQ = 200
DIM = 16
TOTAL = BATCH * SEQ

_NUM_CORES = 2
_NUM_SUBCORES = 16
_NW = _NUM_CORES * _NUM_SUBCORES
_PER_W = TOTAL // _NW
_CHUNK = 2048
_NBUF = 2
_NCHUNKS = _PER_W // _CHUNK
_NROUNDS = _NCHUNKS // _NBUF


def _sc_gather(idx_flat, table):
    mesh = plsc.VectorSubcoreMesh(
        core_axis_name="c", subcore_axis_name="s", num_cores=_NUM_CORES
    )

    @functools.partial(
        pl.kernel,
        out_type=jax.ShapeDtypeStruct((TOTAL, DIM), jnp.float32),
        mesh=mesh,
        scratch_types=[
            pltpu.VMEM((_NBUF, _CHUNK), jnp.int32),
            pltpu.VMEM((_NBUF, _CHUNK, DIM), jnp.float32),
            pltpu.SemaphoreType.DMA((_NBUF,)),
            pltpu.SemaphoreType.DMA((_NBUF,)),
            pltpu.SemaphoreType.DMA((_NBUF,)),
        ],
        compiler_params=pltpu.CompilerParams(use_tc_tiling_on_sc=False),
    )
    def k(idx_hbm, table_hbm, out_hbm, idx_v, rows_v, sem_i, sem_g, sem_o):
        wid = lax.axis_index("s") * _NUM_CORES + lax.axis_index("c")
        base = wid * _PER_W

        for b in range(_NBUF):
            pltpu.async_copy(
                idx_hbm.at[pl.ds(base + b * _CHUNK, _CHUNK)],
                idx_v.at[b],
                sem_i.at[b],
            )

        def round_body(r, carry):
            for b in range(_NBUF):
                off = base + (r * _NBUF + b) * _CHUNK
                pltpu.make_async_copy(
                    idx_hbm.at[pl.ds(off, _CHUNK)], idx_v.at[b], sem_i.at[b]
                ).wait()

                @pl.when(r > 0)
                def _():
                    pltpu.make_async_copy(
                        rows_v.at[b],
                        out_hbm.at[pl.ds(base, _CHUNK)],
                        sem_o.at[b],
                    ).wait()

                pltpu.async_copy(table_hbm.at[idx_v.at[b]], rows_v.at[b], sem_g.at[b])

            for b in range(_NBUF):
                off = base + (r * _NBUF + b) * _CHUNK
                pltpu.make_async_copy(
                    table_hbm.at[idx_v.at[b]], rows_v.at[b], sem_g.at[b]
                ).wait()
                pltpu.async_copy(rows_v.at[b], out_hbm.at[pl.ds(off, _CHUNK)], sem_o.at[b])

                @pl.when(r < _NROUNDS - 1)
                def _():
                    pltpu.async_copy(
                        idx_hbm.at[pl.ds(off + _NBUF * _CHUNK, _CHUNK)],
                        idx_v.at[b],
                        sem_i.at[b],
                    )

            return carry

        lax.fori_loop(0, _NROUNDS, round_body, 0)

        for b in range(_NBUF):
            pltpu.make_async_copy(
                rows_v.at[b], out_hbm.at[pl.ds(base, _CHUNK)], sem_o.at[b]
            ).wait()

    return k(idx_flat, table)


def kernel(positions, position_encoding):
    idx_flat = positions.reshape(TOTAL)
    out = _sc_gather(idx_flat, position_encoding)
    return out.reshape(BATCH, SEQ, DIM)


# speedup vs baseline: 26.5973x; 4.2018x over previous
import functools

import jax
import jax.numpy as jnp
from jax import lax
from jax.experimental import pallas as pl
from jax.experimental.pallas import tpu as pltpu
from jax.experimental.pallas import tpu_sc as plsc

BATCH = 16384
SEQ = 200
DIM = 16
NPOS = 4096

_NUM_CORES = 2
_NUM_SUBCORES = 16
_NW = _NUM_CORES * _NUM_SUBCORES
_BSTRIPE = BATCH // _NW
_BTILES = _BSTRIPE // 128
_SBLK = 8
_NROUNDS = SEQ // _SBLK


def _sc_encode(pos_t, tbl_t):
    mesh = plsc.VectorSubcoreMesh(core_axis_name="c", subcore_axis_name="s")

    @functools.partial(
        pl.kernel,
        out_type=jax.ShapeDtypeStruct(
            (SEQ, 2, BATCH // 128, 8, 128), jnp.float32
        ),
        mesh=mesh,
        scratch_types=[
            pltpu.VMEM((DIM, NPOS), jnp.float32),
            pltpu.VMEM((_SBLK, _BSTRIPE), jnp.int32),
            pltpu.VMEM((2, 2, _BTILES, 8, 128), jnp.float32),
            pltpu.SemaphoreType.DMA((2,)),
        ],
        compiler_params=pltpu.CompilerParams(needs_layout_passes=False),
    )
    def k(pos_hbm, tbl_hbm, out_hbm, tbl_v, pos_v, stg_v, sem_o):
        wid = lax.axis_index("s") * _NUM_CORES + lax.axis_index("c")
        b0 = wid * _BSTRIPE

        pltpu.sync_copy(tbl_hbm, tbl_v)

        def wait_out(sb):
            for kh in range(2):
                pltpu.make_async_copy(
                    stg_v.at[sb, kh],
                    out_hbm.at[0, kh, pl.ds(wid * _BTILES, _BTILES)],
                    sem_o.at[sb],
                ).wait()

        def round_body(r, carry):
            pltpu.sync_copy(
                pos_hbm.at[pl.ds(r * _SBLK, _SBLK), pl.ds(b0, _BSTRIPE)],
                pos_v,
            )
            for j in range(_SBLK):
                sb = j % 2
                s = r * _SBLK + j
                if j >= 2:
                    wait_out(sb)
                else:

                    @pl.when(r > 0)
                    def _():
                        wait_out(sb)

                def g_body(g, c):
                    p = pos_v[j, pl.ds(g * 16, 16)]
                    bh = g // 8
                    bl = (g % 8) * 16
                    for kk in range(DIM):
                        v = plsc.load_gather(
                            tbl_v, [jnp.full((16,), kk, jnp.int32), p]
                        )
                        stg_v[sb, kk // 8, bh, kk % 8, pl.ds(bl, 16)] = v
                    return c

                lax.fori_loop(0, _BSTRIPE // 16, g_body, 0)
                for kh in range(2):
                    pltpu.async_copy(
                        stg_v.at[sb, kh],
                        out_hbm.at[s, kh, pl.ds(wid * _BTILES, _BTILES)],
                        sem_o.at[sb],
                    )
            return carry

        lax.fori_loop(0, _NROUNDS, round_body, 0)
        for sb in range(2):
            wait_out(sb)

    return k(pos_t, tbl_t)


def kernel(positions, position_encoding):
    pos_t = positions.T
    tbl_t = position_encoding.T
    x = _sc_encode(pos_t, tbl_t)
    return x.transpose(2, 4, 0, 1, 3).reshape(BATCH, SEQ, DIM)


# speedup vs baseline: 69.6550x; 2.6189x over previous
import functools

import jax
import jax.numpy as jnp
from jax import lax
from jax.experimental import pallas as pl
from jax.experimental.pallas import tpu as pltpu
from jax.experimental.pallas import tpu_sc as plsc

BATCH = 16384
SEQ = 200
DIM = 16
NPOS = 4096

_NUM_CORES = 2
_NUM_SUBCORES = 16
_NW = _NUM_CORES * _NUM_SUBCORES
_BSTRIPE = BATCH // _NW
_BTILES = _BSTRIPE // 128
_SBLK = 8
_NROUNDS = SEQ // _SBLK


def _sc_encode(pos_t, tbl_t):
    mesh = plsc.VectorSubcoreMesh(core_axis_name="c", subcore_axis_name="s")

    @functools.partial(
        pl.kernel,
        out_type=jax.ShapeDtypeStruct(
            (SEQ, 2, BATCH // 128, 8, 128), jnp.float32
        ),
        mesh=mesh,
        scratch_types=[
            pltpu.VMEM((DIM, NPOS), jnp.float32),
            pltpu.VMEM((_SBLK, _BSTRIPE), jnp.int32),
            pltpu.VMEM((2, 2, _BTILES, 8, 128), jnp.float32),
            pltpu.SemaphoreType.DMA((2,)),
        ],
        compiler_params=pltpu.CompilerParams(needs_layout_passes=False),
    )
    def k(pos_hbm, tbl_hbm, out_hbm, tbl_v, pos_v, stg_v, sem_o):
        wid = lax.axis_index("s") * _NUM_CORES + lax.axis_index("c")
        b0 = wid * _BSTRIPE

        pltpu.sync_copy(tbl_hbm, tbl_v)

        def wait_out(sb):
            for kh in range(2):
                pltpu.make_async_copy(
                    stg_v.at[sb, kh],
                    out_hbm.at[0, kh, pl.ds(wid * _BTILES, _BTILES)],
                    sem_o.at[sb],
                ).wait()

        def round_body(r, carry):
            pltpu.sync_copy(
                pos_hbm.at[pl.ds(r * _SBLK, _SBLK), pl.ds(b0, _BSTRIPE)],
                pos_v,
            )
            for j in range(_SBLK):
                sb = j % 2
                s = r * _SBLK + j
                if j >= 2:
                    wait_out(sb)
                else:

                    @pl.when(r > 0)
                    def _():
                        wait_out(sb)

                @plsc.parallel_loop(0, _BSTRIPE // 16, unroll=4)
                def g_body(g):
                    p = pos_v[j, pl.ds(g * 16, 16)]
                    bh = g // 8
                    bl = (g % 8) * 16
                    for kk in range(DIM):
                        v = plsc.load_gather(
                            tbl_v, [jnp.full((16,), kk, jnp.int32), p]
                        )
                        stg_v[sb, kk // 8, bh, kk % 8, pl.ds(bl, 16)] = v
                for kh in range(2):
                    pltpu.async_copy(
                        stg_v.at[sb, kh],
                        out_hbm.at[s, kh, pl.ds(wid * _BTILES, _BTILES)],
                        sem_o.at[sb],
                    )
            return carry

        lax.fori_loop(0, _NROUNDS, round_body, 0)
        for sb in range(2):
            wait_out(sb)

    return k(pos_t, tbl_t)


def kernel(positions, position_encoding):
    pos_t = positions.T
    tbl_t = position_encoding.T
    x = _sc_encode(pos_t, tbl_t)
    return x.transpose(2, 4, 0, 1, 3).reshape(BATCH, SEQ, DIM)


# speedup vs baseline: 78.7566x; 1.1307x over previous
import functools

import jax
import jax.numpy as jnp
from jax import lax
from jax.experimental import pallas as pl
from jax.experimental.pallas import tpu as pltpu
from jax.experimental.pallas import tpu_sc as plsc

BATCH = 16384
SEQ = 200
DIM = 16
NPOS = 4096

_NUM_CORES = 2
_NUM_SUBCORES = 16
_NW = _NUM_CORES * _NUM_SUBCORES
_BSTRIPE = BATCH // _NW
_BTILES = _BSTRIPE // 128
_SBLK = 4
_NROUNDS = SEQ // _SBLK


def _sc_encode(pos_t, tbl_t):
    mesh = plsc.VectorSubcoreMesh(core_axis_name="c", subcore_axis_name="s")

    @functools.partial(
        pl.kernel,
        out_type=jax.ShapeDtypeStruct(
            (SEQ, 2, BATCH // 128, 8, 128), jnp.float32
        ),
        mesh=mesh,
        scratch_types=[
            pltpu.VMEM((DIM, NPOS), jnp.float32),
            pltpu.VMEM((2, _SBLK, _BSTRIPE), jnp.int32),
            pltpu.VMEM((2, 2, _BTILES, 8, 128), jnp.float32),
            pltpu.SemaphoreType.DMA((2,)),
            pltpu.SemaphoreType.DMA((2,)),
        ],
        compiler_params=pltpu.CompilerParams(needs_layout_passes=False),
    )
    def k(pos_hbm, tbl_hbm, out_hbm, tbl_v, pos_v, stg_v, sem_o, sem_p):
        wid = lax.axis_index("s") * _NUM_CORES + lax.axis_index("c")
        b0 = wid * _BSTRIPE

        def pos_copy(r, pb):
            return pltpu.make_async_copy(
                pos_hbm.at[pl.ds(r * _SBLK, _SBLK), pl.ds(b0, _BSTRIPE)],
                pos_v.at[pb],
                sem_p.at[pb],
            )

        def wait_out(sb):
            for kh in range(2):
                pltpu.make_async_copy(
                    stg_v.at[sb, kh],
                    out_hbm.at[0, kh, pl.ds(wid * _BTILES, _BTILES)],
                    sem_o.at[sb],
                ).wait()

        for pb in range(2):
            pos_copy(pb, pb).start()
        pltpu.sync_copy(tbl_hbm, tbl_v)

        def outer_body(rr, carry):
            for pb in range(2):
                r = 2 * rr + pb
                pos_copy(r, pb).wait()
                for j in range(_SBLK):
                    sb = j % 2
                    s = r * _SBLK + j
                    if j >= 2 or pb == 1:
                        wait_out(sb)
                    else:

                        @pl.when(rr > 0)
                        def _():
                            wait_out(sb)

                    @plsc.parallel_loop(0, _BSTRIPE // 16, unroll=4)
                    def g_body(g):
                        p = pos_v[pb, j, pl.ds(g * 16, 16)]
                        bh = g // 8
                        bl = (g % 8) * 16
                        for kk in range(DIM):
                            v = plsc.load_gather(
                                tbl_v, [jnp.full((16,), kk, jnp.int32), p]
                            )
                            stg_v[sb, kk // 8, bh, kk % 8, pl.ds(bl, 16)] = v
                    for kh in range(2):
                        pltpu.async_copy(
                            stg_v.at[sb, kh],
                            out_hbm.at[s, kh, pl.ds(wid * _BTILES, _BTILES)],
                            sem_o.at[sb],
                        )

                @pl.when(r < _NROUNDS - 2)
                def _():
                    pos_copy(r + 2, pb).start()

            return carry

        lax.fori_loop(0, _NROUNDS // 2, outer_body, 0)
        for sb in range(2):
            wait_out(sb)

    return k(pos_t, tbl_t)


def kernel(positions, position_encoding):
    pos_t = positions.T
    tbl_t = position_encoding.T
    x = _sc_encode(pos_t, tbl_t)
    return x.transpose(2, 4, 0, 1, 3).reshape(BATCH, SEQ, DIM)
